# baseline (speedup 1.0000x reference)
import functools

import jax
import jax.numpy as jnp
from jax import lax
from jax.experimental import pallas as pl
from jax.experimental.pallas import tpu as pltpu
from jax.experimental.pallas import tpu_sc as plsc

T, D, E, H = 2048, 768, 64, 512
BT = 64
NB = T // BT
NT = NB + E - 1
ML = 128
NW = 32
R = T // NW
DX = D + 128


def _router_body(x_ref, wg_ref, b_ref, xw_ref, p_ref, e_ref, bt_ref, lo_ref,
                 hi_ref):
    x = x_ref[...]
    gates = lax.dot_general(x, wg_ref[...], (((1,), (1,)), ((), ())),
                            preferred_element_type=jnp.float32)
    scores = jax.nn.sigmoid(gates)
    biased = scores + b_ref[...]
    m = jnp.max(biased, axis=1, keepdims=True)
    iota_e = lax.broadcasted_iota(jnp.int32, (T, E), 1)
    e_tok = jnp.min(jnp.where(biased >= m, iota_e, E), axis=1,
                    keepdims=True)
    sel = iota_e == e_tok
    w_tok = jnp.sum(jnp.where(sel, scores, 0.0), axis=1, keepdims=True)
    xw_ref[:, :D] = x
    xw_ref[:, D:] = jnp.broadcast_to(w_tok, (T, DX - D))

    oh = sel.astype(jnp.bfloat16)
    lt_t = (lax.broadcasted_iota(jnp.int32, (T, T), 0) >
            lax.broadcasted_iota(jnp.int32, (T, T), 1)).astype(jnp.bfloat16)
    excl = lax.dot_general(lt_t, oh, (((1,), (0,)), ((), ())),
                           preferred_element_type=jnp.float32)
    counts = jnp.sum(sel.astype(jnp.float32), axis=0, keepdims=True)
    lt_e = (lax.broadcasted_iota(jnp.int32, (E, E), 0) <
            lax.broadcasted_iota(jnp.int32, (E, E), 1)).astype(jnp.float32)
    offs = lax.dot_general(counts, lt_e, (((1,), (0,)), ((), ())),
                           preferred_element_type=jnp.float32,
                           precision=lax.Precision.HIGHEST)
    p = jnp.sum(jnp.where(sel, excl + offs, 0.0), axis=1, keepdims=True)
    p_ref[...] = p.astype(jnp.int32)

    start = offs
    end = offs + counts
    nonempty = counts > 0.0
    fb = jnp.floor(start * (1.0 / BT))
    lb = jnp.floor((end - 1.0) * (1.0 / BT))
    nbl = jnp.where(nonempty, lb - fb + 1.0, 0.0)
    tstart = lax.dot_general(nbl, lt_e, (((1,), (0,)), ((), ())),
                             preferred_element_type=jnp.float32,
                             precision=lax.Precision.HIGHEST)
    cend = tstart + nbl
    total = tstart[0, E - 1] + nbl[0, E - 1]

    i_col = lax.broadcasted_iota(jnp.int32, (ML, 1), 0).astype(jnp.float32)
    i_eff = jnp.minimum(i_col, total - 1.0)
    owner = jnp.sum((cend <= i_eff).astype(jnp.float32), axis=1,
                    keepdims=True)
    oh_t = (lax.broadcasted_iota(jnp.int32, (ML, E), 1)
            .astype(jnp.float32) == owner)

    def pick(v):
        return jnp.sum(jnp.where(oh_t, v, 0.0), axis=1, keepdims=True)

    st_o = pick(start)
    en_o = pick(end)
    b_t = pick(fb) + (i_eff - pick(tstart))
    blo = b_t * BT
    lo = jnp.maximum(st_o, blo) - blo
    hi = jnp.minimum(en_o, blo + BT) - blo
    valid = i_col < total
    lo = jnp.where(valid, lo, 0.0)
    hi = jnp.where(valid, hi, 0.0)
    e_ref[...] = owner.astype(jnp.int32)
    bt_ref[...] = b_t.astype(jnp.int32)
    lo_ref[...] = lo.astype(jnp.int32)
    hi_ref[...] = hi.astype(jnp.int32)


def _router(x, Wg, bias):
    out_shape = [
        jax.ShapeDtypeStruct((T, DX), jnp.float32),
        jax.ShapeDtypeStruct((T, 1), jnp.int32),
        jax.ShapeDtypeStruct((ML, 1), jnp.int32),
        jax.ShapeDtypeStruct((ML, 1), jnp.int32),
        jax.ShapeDtypeStruct((ML, 1), jnp.int32),
        jax.ShapeDtypeStruct((ML, 1), jnp.int32),
    ]
    return pl.pallas_call(_router_body, out_shape=out_shape)(
        x, Wg, bias.reshape(1, E))


def _sc_mesh():
    return plsc.VectorSubcoreMesh(core_axis_name="c", subcore_axis_name="s",
                                  num_cores=2, num_subcores=16)


def _sc_wid():
    return lax.axis_index("s") * 2 + lax.axis_index("c")


def _scatter_body(xe_hbm, p_hbm, xs_hbm, idx_v, rows_v, sem):
    base = _sc_wid() * R
    pltpu.sync_copy(p_hbm.at[pl.ds(base, R)], idx_v)
    pltpu.sync_copy(xe_hbm.at[pl.ds(base, R)], rows_v)
    pltpu.async_copy(rows_v, xs_hbm.at[idx_v], sem).wait()


@functools.cache
def _sc_scatter():
    return pl.kernel(
        _scatter_body,
        out_type=jax.ShapeDtypeStruct((T, DX), jnp.float32),
        mesh=_sc_mesh(),
        scratch_types=[pltpu.VMEM((R,), jnp.int32),
                       pltpu.VMEM((R, DX), jnp.float32),
                       pltpu.SemaphoreType.DMA],
    )


def _gather_body(ys_hbm, p_hbm, out_hbm, idx_v, rows_v, sem):
    base = _sc_wid() * R
    pltpu.sync_copy(p_hbm.at[pl.ds(base, R)], idx_v)
    pltpu.async_copy(ys_hbm.at[idx_v], rows_v, sem).wait()
    pltpu.sync_copy(rows_v, out_hbm.at[pl.ds(base, R)])


@functools.cache
def _sc_gather():
    return pl.kernel(
        _gather_body,
        out_type=jax.ShapeDtypeStruct((T, D), jnp.float32),
        mesh=_sc_mesh(),
        scratch_types=[pltpu.VMEM((R,), jnp.int32),
                       pltpu.VMEM((R, D), jnp.float32),
                       pltpu.SemaphoreType.DMA],
    )


def _gmm_body(e_ref, b_ref, lo_ref, hi_ref, xs_ref, wg_ref, wu_ref,
              wd_ref, out_ref):
    i = pl.program_id(0)
    lo = lo_ref[i]
    hi = hi_ref[i]

    @pl.when(hi > lo)
    def _():
        xb = xs_ref[:, :D]
        ws = xs_ref[:, D:D + 1]
        g = lax.dot_general(xb, wg_ref[0], (((1,), (1,)), ((), ())),
                            preferred_element_type=jnp.float32)
        u = lax.dot_general(xb, wu_ref[0], (((1,), (1,)), ((), ())),
                            preferred_element_type=jnp.float32)
        h = g * jax.nn.sigmoid(g) * u
        y = lax.dot_general(h, wd_ref[0], (((1,), (1,)), ((), ())),
                            preferred_element_type=jnp.float32)
        r = lax.broadcasted_iota(jnp.int32, (BT, 1), 0)
        keep = (r >= lo) & (r < hi)
        out_ref[...] = jnp.where(keep, y * ws, out_ref[...])


def _gmm(e_t, b_t, lo_t, hi_t, xs, Wgate, Wup, Wdown):
    grid_spec = pltpu.PrefetchScalarGridSpec(
        num_scalar_prefetch=4,
        grid=(NT,),
        in_specs=[
            pl.BlockSpec((BT, DX), lambda i, e, b, lo, hi: (b[i], 0)),
            pl.BlockSpec((1, H, D), lambda i, e, b, lo, hi: (e[i], 0, 0)),
            pl.BlockSpec((1, H, D), lambda i, e, b, lo, hi: (e[i], 0, 0)),
            pl.BlockSpec((1, D, H), lambda i, e, b, lo, hi: (e[i], 0, 0)),
        ],
        out_specs=pl.BlockSpec((BT, D), lambda i, e, b, lo, hi: (b[i], 0)),
    )
    return pl.pallas_call(
        _gmm_body,
        grid_spec=grid_spec,
        out_shape=jax.ShapeDtypeStruct((T, D), jnp.float32),
    )(e_t, b_t, lo_t, hi_t, xs, Wgate, Wup, Wdown)


def kernel(x, Wg, bias, Wgate, Wup, Wdown):
    xe, p2, e_t, b_t, lo_t, hi_t = _router(x, Wg, bias)
    p = p2.reshape(T)
    xs = _sc_scatter()(xe, p)
    ys = _gmm(e_t.reshape(ML), b_t.reshape(ML), lo_t.reshape(ML),
              hi_t.reshape(ML), xs, Wgate, Wup, Wdown)
    return _sc_gather()(ys, p)
    p = p2.reshape(T)
    xs, xsw = _sc_scatter()(x, xw, p)
    ys = _gmm(e_t.reshape(ML), b_t.reshape(ML), lo_t.reshape(ML),
              hi_t.reshape(ML), xs, xsw, Wgate, Wup, Wdown)
    return _sc_gather()(ys, p)


# speedup vs baseline: 1.1611x; 1.1611x over previous
import functools

import jax
import jax.numpy as jnp
from jax import lax
from jax.experimental import pallas as pl
from jax.experimental.pallas import tpu as pltpu
from jax.experimental.pallas import tpu_sc as plsc

T, D, E, H = 2048, 768, 64, 512
BT = 256
NB = T // BT
NT = NB + E - 1
ML = 128
NW = 32
R = T // NW
DX = D + 128


def _router_body(x_ref, wg_ref, b_ref, xw_ref, p_ref, e_ref, bt_ref, lo_ref,
                 hi_ref):
    x = x_ref[...]
    gates = lax.dot_general(x, wg_ref[...], (((1,), (1,)), ((), ())),
                            preferred_element_type=jnp.float32)
    scores = jax.nn.sigmoid(gates)
    biased = scores + b_ref[...]
    m = jnp.max(biased, axis=1, keepdims=True)
    iota_e = lax.broadcasted_iota(jnp.int32, (T, E), 1)
    e_tok = jnp.min(jnp.where(biased >= m, iota_e, E), axis=1,
                    keepdims=True)
    sel = iota_e == e_tok
    w_tok = jnp.sum(jnp.where(sel, scores, 0.0), axis=1, keepdims=True)
    xw_ref[:, :D] = x
    xw_ref[:, D:] = jnp.broadcast_to(w_tok, (T, DX - D))

    oh = sel.astype(jnp.bfloat16)
    lt_t = (lax.broadcasted_iota(jnp.int32, (T, T), 0) >
            lax.broadcasted_iota(jnp.int32, (T, T), 1)).astype(jnp.bfloat16)
    excl = lax.dot_general(lt_t, oh, (((1,), (0,)), ((), ())),
                           preferred_element_type=jnp.float32)
    counts = jnp.sum(sel.astype(jnp.float32), axis=0, keepdims=True)
    lt_e = (lax.broadcasted_iota(jnp.int32, (E, E), 0) <
            lax.broadcasted_iota(jnp.int32, (E, E), 1)).astype(jnp.float32)
    offs = lax.dot_general(counts, lt_e, (((1,), (0,)), ((), ())),
                           preferred_element_type=jnp.float32,
                           precision=lax.Precision.HIGHEST)
    p = jnp.sum(jnp.where(sel, excl + offs, 0.0), axis=1, keepdims=True)
    p_ref[...] = p.astype(jnp.int32)

    start = offs
    end = offs + counts
    nonempty = counts > 0.0
    fb = jnp.floor(start * (1.0 / BT))
    lb = jnp.floor((end - 1.0) * (1.0 / BT))
    nbl = jnp.where(nonempty, lb - fb + 1.0, 0.0)
    tstart = lax.dot_general(nbl, lt_e, (((1,), (0,)), ((), ())),
                             preferred_element_type=jnp.float32,
                             precision=lax.Precision.HIGHEST)
    cend = tstart + nbl
    total = tstart[0, E - 1] + nbl[0, E - 1]

    i_col = lax.broadcasted_iota(jnp.int32, (ML, 1), 0).astype(jnp.float32)
    i_eff = jnp.minimum(i_col, total - 1.0)
    owner = jnp.sum((cend <= i_eff).astype(jnp.float32), axis=1,
                    keepdims=True)
    oh_t = (lax.broadcasted_iota(jnp.int32, (ML, E), 1)
            .astype(jnp.float32) == owner)

    def pick(v):
        return jnp.sum(jnp.where(oh_t, v, 0.0), axis=1, keepdims=True)

    st_o = pick(start)
    en_o = pick(end)
    b_t = pick(fb) + (i_eff - pick(tstart))
    blo = b_t * BT
    lo = jnp.maximum(st_o, blo) - blo
    hi = jnp.minimum(en_o, blo + BT) - blo
    valid = i_col < total
    lo = jnp.where(valid, lo, 0.0)
    hi = jnp.where(valid, hi, 0.0)
    e_ref[...] = owner.astype(jnp.int32)
    bt_ref[...] = b_t.astype(jnp.int32)
    lo_ref[...] = lo.astype(jnp.int32)
    hi_ref[...] = hi.astype(jnp.int32)


def _router(x, Wg, bias):
    out_shape = [
        jax.ShapeDtypeStruct((T, DX), jnp.float32),
        jax.ShapeDtypeStruct((T, 1), jnp.int32),
        jax.ShapeDtypeStruct((ML, 1), jnp.int32),
        jax.ShapeDtypeStruct((ML, 1), jnp.int32),
        jax.ShapeDtypeStruct((ML, 1), jnp.int32),
        jax.ShapeDtypeStruct((ML, 1), jnp.int32),
    ]
    return pl.pallas_call(_router_body, out_shape=out_shape)(
        x, Wg, bias.reshape(1, E))


def _sc_mesh():
    return plsc.VectorSubcoreMesh(core_axis_name="c", subcore_axis_name="s",
                                  num_cores=2, num_subcores=16)


def _sc_wid():
    return lax.axis_index("s") * 2 + lax.axis_index("c")


def _scatter_body(xe_hbm, p_hbm, xs_hbm, idx_v, rows_v, sem):
    base = _sc_wid() * R
    pltpu.sync_copy(p_hbm.at[pl.ds(base, R)], idx_v)
    pltpu.sync_copy(xe_hbm.at[pl.ds(base, R)], rows_v)
    pltpu.async_copy(rows_v, xs_hbm.at[idx_v], sem).wait()


@functools.cache
def _sc_scatter():
    return pl.kernel(
        _scatter_body,
        out_type=jax.ShapeDtypeStruct((T, DX), jnp.float32),
        mesh=_sc_mesh(),
        scratch_types=[pltpu.VMEM((R,), jnp.int32),
                       pltpu.VMEM((R, DX), jnp.float32),
                       pltpu.SemaphoreType.DMA],
    )


def _gather_body(ys_hbm, p_hbm, out_hbm, idx_v, rows_v, sem):
    base = _sc_wid() * R
    pltpu.sync_copy(p_hbm.at[pl.ds(base, R)], idx_v)
    pltpu.async_copy(ys_hbm.at[idx_v], rows_v, sem).wait()
    pltpu.sync_copy(rows_v, out_hbm.at[pl.ds(base, R)])


@functools.cache
def _sc_gather():
    return pl.kernel(
        _gather_body,
        out_type=jax.ShapeDtypeStruct((T, D), jnp.float32),
        mesh=_sc_mesh(),
        scratch_types=[pltpu.VMEM((R,), jnp.int32),
                       pltpu.VMEM((R, D), jnp.float32),
                       pltpu.SemaphoreType.DMA],
    )


def _gmm_body(e_ref, b_ref, lo_ref, hi_ref, xs_ref, wg_ref, wu_ref,
              wd_ref, out_ref):
    i = pl.program_id(0)
    lo = lo_ref[i]
    hi = hi_ref[i]

    @pl.when(hi > lo)
    def _():
        xb = xs_ref[:, :D]
        ws = xs_ref[:, D:D + 1]
        g = lax.dot_general(xb, wg_ref[0], (((1,), (1,)), ((), ())),
                            preferred_element_type=jnp.float32)
        u = lax.dot_general(xb, wu_ref[0], (((1,), (1,)), ((), ())),
                            preferred_element_type=jnp.float32)
        h = g * jax.nn.sigmoid(g) * u
        y = lax.dot_general(h, wd_ref[0], (((1,), (1,)), ((), ())),
                            preferred_element_type=jnp.float32)
        r = lax.broadcasted_iota(jnp.int32, (BT, 1), 0)
        keep = (r >= lo) & (r < hi)
        out_ref[...] = jnp.where(keep, y * ws, out_ref[...])


def _gmm(e_t, b_t, lo_t, hi_t, xs, Wgate, Wup, Wdown):
    grid_spec = pltpu.PrefetchScalarGridSpec(
        num_scalar_prefetch=4,
        grid=(NT,),
        in_specs=[
            pl.BlockSpec((BT, DX), lambda i, e, b, lo, hi: (b[i], 0)),
            pl.BlockSpec((1, H, D), lambda i, e, b, lo, hi: (e[i], 0, 0)),
            pl.BlockSpec((1, H, D), lambda i, e, b, lo, hi: (e[i], 0, 0)),
            pl.BlockSpec((1, D, H), lambda i, e, b, lo, hi: (e[i], 0, 0)),
        ],
        out_specs=pl.BlockSpec((BT, D), lambda i, e, b, lo, hi: (b[i], 0)),
    )
    return pl.pallas_call(
        _gmm_body,
        grid_spec=grid_spec,
        out_shape=jax.ShapeDtypeStruct((T, D), jnp.float32),
    )(e_t, b_t, lo_t, hi_t, xs, Wgate, Wup, Wdown)


def kernel(x, Wg, bias, Wgate, Wup, Wdown):
    xe, p2, e_t, b_t, lo_t, hi_t = _router(x, Wg, bias)
    p = p2.reshape(T)
    xs = _sc_scatter()(xe, p)
    ys = _gmm(e_t.reshape(ML), b_t.reshape(ML), lo_t.reshape(ML),
              hi_t.reshape(ML), xs, Wgate, Wup, Wdown)
    return _sc_gather()(ys, p)
    p = p2.reshape(T)
    xs, xsw = _sc_scatter()(x, xw, p)
    ys = _gmm(e_t.reshape(ML), b_t.reshape(ML), lo_t.reshape(ML),
              hi_t.reshape(ML), xs, xsw, Wgate, Wup, Wdown)
    return _sc_gather()(ys, p)


# speedup vs baseline: 1.3916x; 1.1985x over previous
import functools

import jax
import jax.numpy as jnp
from jax import lax
from jax.experimental import pallas as pl
from jax.experimental.pallas import tpu as pltpu
from jax.experimental.pallas import tpu_sc as plsc

T, D, E, H = 2048, 768, 64, 512
BT = 128
NB = T // BT
NT = NB + E - 1
ML = 128
NW = 32
R = T // NW
DX = D + 128
NBUF = 4


def _router_body(x_ref, wg_ref, b_ref, xw_ref, p_ref, bt_ref, lo_ref,
                 hi_ref, f_ref, ne_ref, eu_ref, sz_ref):
    x = x_ref[...]
    gates = lax.dot_general(x, wg_ref[...], (((1,), (1,)), ((), ())),
                            preferred_element_type=jnp.float32)
    scores = jax.nn.sigmoid(gates)
    biased = scores + b_ref[...]
    m = jnp.max(biased, axis=1, keepdims=True)
    iota_e = lax.broadcasted_iota(jnp.int32, (T, E), 1)
    e_tok = jnp.min(jnp.where(biased >= m, iota_e, E), axis=1,
                    keepdims=True)
    sel = iota_e == e_tok
    w_tok = jnp.sum(jnp.where(sel, scores, 0.0), axis=1, keepdims=True)
    xw_ref[:, :D] = x
    xw_ref[:, D:] = jnp.broadcast_to(w_tok, (T, DX - D))

    oh = sel.astype(jnp.bfloat16)
    lt_t = (lax.broadcasted_iota(jnp.int32, (T, T), 0) >
            lax.broadcasted_iota(jnp.int32, (T, T), 1)).astype(jnp.bfloat16)
    excl = lax.dot_general(lt_t, oh, (((1,), (0,)), ((), ())),
                           preferred_element_type=jnp.float32)
    counts = jnp.sum(sel.astype(jnp.float32), axis=0, keepdims=True)
    lt_e = (lax.broadcasted_iota(jnp.int32, (E, E), 0) <
            lax.broadcasted_iota(jnp.int32, (E, E), 1)).astype(jnp.float32)
    offs = lax.dot_general(counts, lt_e, (((1,), (0,)), ((), ())),
                           preferred_element_type=jnp.float32,
                           precision=lax.Precision.HIGHEST)
    p = jnp.sum(jnp.where(sel, excl + offs, 0.0), axis=1, keepdims=True)
    p_ref[...] = p.astype(jnp.int32)

    start = offs
    end = offs + counts
    nonempty = counts > 0.0
    fb = jnp.floor(start * (1.0 / BT))
    lb = jnp.floor((end - 1.0) * (1.0 / BT))
    nbl = jnp.where(nonempty, lb - fb + 1.0, 0.0)
    tstart = lax.dot_general(nbl, lt_e, (((1,), (0,)), ((), ())),
                             preferred_element_type=jnp.float32,
                             precision=lax.Precision.HIGHEST)
    cend = tstart + nbl
    total = tstart[0, E - 1] + nbl[0, E - 1]

    i_col = lax.broadcasted_iota(jnp.int32, (ML, 1), 0).astype(jnp.float32)
    i_eff = jnp.minimum(i_col, total - 1.0)
    owner = jnp.sum((cend <= i_eff).astype(jnp.float32), axis=1,
                    keepdims=True)
    oh_t = (lax.broadcasted_iota(jnp.int32, (ML, E), 1)
            .astype(jnp.float32) == owner)

    def pick(v):
        return jnp.sum(jnp.where(oh_t, v, 0.0), axis=1, keepdims=True)

    st_o = pick(start)
    en_o = pick(end)
    b_t = pick(fb) + (i_eff - pick(tstart))
    blo = b_t * BT
    lo = jnp.maximum(st_o, blo) - blo
    hi = jnp.minimum(en_o, blo + BT) - blo
    valid = i_col < total
    lo = jnp.where(valid, lo, 0.0)
    hi = jnp.where(valid, hi, 0.0)
    bt_ref[...] = b_t.astype(jnp.int32)
    lo_ref[...] = lo.astype(jnp.int32)
    hi_ref[...] = hi.astype(jnp.int32)

    used = nonempty.astype(jnp.float32)
    le_e = (lax.broadcasted_iota(jnp.int32, (E, E), 0) <=
            lax.broadcasted_iota(jnp.int32, (E, E), 1)).astype(jnp.float32)
    ucum = lax.dot_general(used, le_e, (((1,), (0,)), ((), ())),
                           preferred_element_type=jnp.float32,
                           precision=lax.Precision.HIGHEST)
    f_ref[...] = (pick(ucum) - 1.0).astype(jnp.int32)
    newe = (i_col == pick(tstart)) & valid
    ne_ref[...] = newe.astype(jnp.int32)
    j_col = lax.broadcasted_iota(jnp.int32, (E, 1), 0).astype(jnp.float32)
    iota_ee = lax.broadcasted_iota(jnp.int32, (E, E), 1).astype(jnp.float32)
    m_ue = ((ucum - 1.0) == j_col) & (used > 0.0)
    eu_ref[...] = jnp.sum(jnp.where(m_ue, iota_ee, 0.0), axis=1,
                          keepdims=True).astype(jnp.int32)
    nu = jnp.sum(used)
    i8 = lax.broadcasted_iota(jnp.int32, (8, 1), 0).astype(jnp.float32)
    sz_ref[...] = jnp.where(i8 == 0.0, total, nu).astype(jnp.int32)


def _router(x, Wg, bias):
    out_shape = [
        jax.ShapeDtypeStruct((T, DX), jnp.float32),
        jax.ShapeDtypeStruct((T, 1), jnp.int32),
        jax.ShapeDtypeStruct((ML, 1), jnp.int32),
        jax.ShapeDtypeStruct((ML, 1), jnp.int32),
        jax.ShapeDtypeStruct((ML, 1), jnp.int32),
        jax.ShapeDtypeStruct((ML, 1), jnp.int32),
        jax.ShapeDtypeStruct((ML, 1), jnp.int32),
        jax.ShapeDtypeStruct((E, 1), jnp.int32),
        jax.ShapeDtypeStruct((8, 1), jnp.int32),
    ]
    return pl.pallas_call(_router_body, out_shape=out_shape)(
        x, Wg, bias.reshape(1, E))


def _sc_mesh():
    return plsc.VectorSubcoreMesh(core_axis_name="c", subcore_axis_name="s",
                                  num_cores=2, num_subcores=16)


def _sc_wid():
    return lax.axis_index("s") * 2 + lax.axis_index("c")


def _scatter_body(xe_hbm, p_hbm, xs_hbm, idx_v, rows_v, sem):
    base = _sc_wid() * R
    pltpu.sync_copy(p_hbm.at[pl.ds(base, R)], idx_v)
    pltpu.sync_copy(xe_hbm.at[pl.ds(base, R)], rows_v)
    pltpu.async_copy(rows_v, xs_hbm.at[idx_v], sem).wait()


@functools.cache
def _sc_scatter():
    return pl.kernel(
        _scatter_body,
        out_type=jax.ShapeDtypeStruct((T, DX), jnp.float32),
        mesh=_sc_mesh(),
        scratch_types=[pltpu.VMEM((R,), jnp.int32),
                       pltpu.VMEM((R, DX), jnp.float32),
                       pltpu.SemaphoreType.DMA],
    )


def _gather_body(ys_hbm, p_hbm, out_hbm, idx_v, rows_v, sem):
    base = _sc_wid() * R
    pltpu.sync_copy(p_hbm.at[pl.ds(base, R)], idx_v)
    pltpu.async_copy(ys_hbm.at[idx_v], rows_v, sem).wait()
    pltpu.sync_copy(rows_v, out_hbm.at[pl.ds(base, R)])


@functools.cache
def _sc_gather():
    return pl.kernel(
        _gather_body,
        out_type=jax.ShapeDtypeStruct((T, D), jnp.float32),
        mesh=_sc_mesh(),
        scratch_types=[pltpu.VMEM((R,), jnp.int32),
                       pltpu.VMEM((R, D), jnp.float32),
                       pltpu.SemaphoreType.DMA],
    )


def _gmm_body(bt_s, lo_s, hi_s, f_s, ne_s, eu_s, sz_s,
              xs_any, wg_any, wu_any, wd_any, ys_ref,
              xs_v, wgb, wub, wdb, sem_x, sem_w):
    ntiles = sz_s[0]
    nu = sz_s[1]

    def fetch(j):
        slot = lax.rem(j, NBUF)
        e = eu_s[j]
        pltpu.make_async_copy(wg_any.at[e], wgb.at[slot],
                              sem_w.at[slot, 0]).start()
        pltpu.make_async_copy(wu_any.at[e], wub.at[slot],
                              sem_w.at[slot, 1]).start()
        pltpu.make_async_copy(wd_any.at[e], wdb.at[slot],
                              sem_w.at[slot, 2]).start()

    def wait_slot(j):
        slot = lax.rem(j, NBUF)
        e = eu_s[j]
        pltpu.make_async_copy(wg_any.at[e], wgb.at[slot],
                              sem_w.at[slot, 0]).wait()
        pltpu.make_async_copy(wu_any.at[e], wub.at[slot],
                              sem_w.at[slot, 1]).wait()
        pltpu.make_async_copy(wd_any.at[e], wdb.at[slot],
                              sem_w.at[slot, 2]).wait()

    fetch(jnp.int32(0))
    pltpu.make_async_copy(xs_any, xs_v, sem_x).start()
    for j in range(1, NBUF - 1):
        @pl.when(j < nu)
        def _(j=j):
            fetch(jnp.int32(j))
    pltpu.make_async_copy(xs_any, xs_v, sem_x).wait()

    def step(i, carry):
        @pl.when(i < ntiles)
        def _():
            fi = f_s[i]

            @pl.when(ne_s[i] == 1)
            def _():
                @pl.when(fi + (NBUF - 1) < nu)
                def _():
                    fetch(fi + (NBUF - 1))
                wait_slot(fi)

            slot = lax.rem(fi, NBUF)
            boff = bt_s[i] * BT
            xb = xs_v[pl.ds(boff, BT), :D]
            g = lax.dot_general(xb, wgb[slot], (((1,), (1,)), ((), ())),
                                preferred_element_type=jnp.float32)
            u = lax.dot_general(xb, wub[slot], (((1,), (1,)), ((), ())),
                                preferred_element_type=jnp.float32)
            h = g * jax.nn.sigmoid(g) * u
            y = lax.dot_general(h, wdb[slot], (((1,), (1,)), ((), ())),
                                preferred_element_type=jnp.float32)
            ws = xs_v[pl.ds(boff, BT), D:D + 1]
            r = lax.broadcasted_iota(jnp.int32, (BT, 1), 0)
            keep = (r >= lo_s[i]) & (r < hi_s[i])
            ys_ref[pl.ds(boff, BT), :] = jnp.where(
                keep, y * ws, ys_ref[pl.ds(boff, BT), :])
        return carry

    lax.fori_loop(0, NT, step, 0)


def _gmm(bt_t, lo_t, hi_t, f_t, ne_t, eu_t, sz_t, xs, Wgate, Wup, Wdown,
         interpret=False):
    smem = pl.BlockSpec(memory_space=pltpu.SMEM)
    anym = pl.BlockSpec(memory_space=pl.ANY)
    return pl.pallas_call(
        _gmm_body,
        in_specs=[smem, smem, smem, smem, smem, smem, smem,
                  anym, anym, anym, anym],
        out_specs=pl.BlockSpec(memory_space=pltpu.VMEM),
        out_shape=jax.ShapeDtypeStruct((T, D), jnp.float32),
        scratch_shapes=[
            pltpu.VMEM((T, DX), jnp.float32),
            pltpu.VMEM((NBUF, H, D), jnp.float32),
            pltpu.VMEM((NBUF, H, D), jnp.float32),
            pltpu.VMEM((NBUF, D, H), jnp.float32),
            pltpu.SemaphoreType.DMA,
            pltpu.SemaphoreType.DMA((NBUF, 3)),
        ],
        interpret=interpret,
    )(bt_t, lo_t, hi_t, f_t, ne_t, eu_t, sz_t, xs, Wgate, Wup, Wdown)


def kernel(x, Wg, bias, Wgate, Wup, Wdown):
    xe, p2, b_t, lo_t, hi_t, f_t, ne_t, eu_t, sz_t = _router(x, Wg, bias)
    p = p2.reshape(T)
    xs = _sc_scatter()(xe, p)
    ys = _gmm(b_t.reshape(ML), lo_t.reshape(ML), hi_t.reshape(ML),
              f_t.reshape(ML), ne_t.reshape(ML), eu_t.reshape(E),
              sz_t.reshape(8), xs, Wgate, Wup, Wdown)
    return _sc_gather()(ys, p)
    p = p2.reshape(T)
    xs, xsw = _sc_scatter()(x, xw, p)
    ys = _gmm(e_t.reshape(ML), b_t.reshape(ML), lo_t.reshape(ML),
              hi_t.reshape(ML), xs, xsw, Wgate, Wup, Wdown)
    return _sc_gather()(ys, p)
